# baseline (speedup 1.0000x reference)
import functools

import jax
import jax.numpy as jnp
from jax import lax
from jax.experimental import pallas as pl
from jax.experimental.pallas import tpu as pltpu
from jax.experimental.pallas import tpu_sc as plsc

N = 4096
M = 1024
ETA = 0.1
MU_WTS = 2.5

LW_BLK = 128

NW = 32
SLAB = N // NW
CH = 64
MAXCH = (N + CH - 1) // CH
IDX_CAP = N + CH


def _sc_phase1_body(f_hbm, x_hbm, thr_hbm, w2_hbm, nf_hbm,
                    f_v, idx_v, x_v, thr_v, nf_v, acc_v, rows0, rows1,
                    sem0, sem1):
    cid = lax.axis_index("c")
    sid = lax.axis_index("s")
    wid = sid * 2 + cid
    base = wid * SLAB

    pltpu.sync_copy(f_hbm, f_v)
    pltpu.sync_copy(x_hbm.at[pl.ds(base, SLAB)], x_v)
    pltpu.sync_copy(thr_hbm.at[pl.ds(base, SLAB)], thr_v)

    iota = lax.broadcasted_iota(jnp.int32, (16,), 0) * NW

    def compact_body(c, cnt):
        fv = f_v[pl.ds(c * 16, 16)]
        mi = fv.astype(jnp.int32)
        pos = plsc.cumsum(mi)
        ivec = iota + (c * (16 * NW) + wid)
        plsc.store_scatter(idx_v, [pos + (cnt - 1)], ivec, mask=fv > 0.0)
        return cnt + jnp.sum(mi)

    cnt = lax.fori_loop(0, N // 16, compact_body, jnp.int32(0))

    pad = jnp.zeros((16,), jnp.int32)
    for p in range(CH // 16):
        idx_v[pl.ds(cnt + p * 16, 16)] = pad

    nch = (cnt + CH - 1) // CH

    def start_gather(c, buf, sem):
        pltpu.async_copy(w2_hbm.at[idx_v.at[pl.ds(c * CH, CH)]], buf, sem)

    def wait_gather(c, buf, sem):
        pltpu.make_async_copy(w2_hbm.at[idx_v.at[pl.ds(c * CH, CH)]],
                              buf, sem).wait()

    zero16 = jnp.zeros((16,), jnp.float32)
    for k in range(SLAB // 16):
        acc_v[pl.ds(k * 16, 16)] = zero16

    @pl.when(nch > 0)
    def _():
        start_gather(0, rows0, sem0)

    def accumulate(buf, valid):
        for k in range(SLAB // 16):
            def rbody(r, a, k=k):
                return a + buf[r, pl.ds(k * 16, 16)]
            acc = lax.fori_loop(0, valid, rbody,
                                acc_v[pl.ds(k * 16, 16)])
            acc_v[pl.ds(k * 16, 16)] = acc

    def chunk_body(c, carry):
        even = lax.rem(c, 2) == 0

        @pl.when(c + 1 < nch)
        def _():
            @pl.when(even)
            def _():
                start_gather(c + 1, rows1, sem1)

            @pl.when(jnp.logical_not(even))
            def _():
                start_gather(c + 1, rows0, sem0)

        valid = jnp.minimum(cnt - c * CH, CH)

        @pl.when(even)
        def _():
            wait_gather(c, rows0, sem0)
            accumulate(rows0, valid)

        @pl.when(jnp.logical_not(even))
        def _():
            wait_gather(c, rows1, sem1)
            accumulate(rows1, valid)

        return carry

    lax.fori_loop(0, nch, chunk_body, jnp.int32(0))

    ones = jnp.ones((16,), jnp.float32)
    zeros = jnp.zeros((16,), jnp.float32)
    for k in range(SLAB // 16):
        s = pl.ds(k * 16, 16)
        nf_v[s] = jnp.where(acc_v[s] + x_v[s] > thr_v[s], ones, zeros)
    pltpu.sync_copy(nf_v, nf_hbm.at[pl.ds(base, SLAB)])


def _phase2_body(nf_ref, w_ref, thr_ref, wout_ref, act_ref, maxv_ref,
                 maxi_ref, smax, sidx):
    i = pl.program_id(0)
    w = w_ref[...]
    wout_ref[...] = w
    wb = w.astype(jnp.bfloat16).astype(jnp.float32)
    a = lax.dot_general(wb, nf_ref[...],
                        (((1,), (1,)), ((), ())),
                        preferred_element_type=jnp.float32)
    lgn_act = jnp.maximum(a, 0.0)
    act_ref[...] = lgn_act
    actv = jnp.maximum(lgn_act - thr_ref[...], 0.0)
    bmax = jnp.max(actv)
    iota = lax.broadcasted_iota(jnp.int32, (LW_BLK, 1), 0)
    bidx = jnp.min(jnp.where(actv == bmax, iota, 2 ** 30)) + i * LW_BLK

    @pl.when(i == 0)
    def _():
        smax[0] = bmax
        sidx[0] = bidx

    @pl.when(i > 0)
    def _():
        better = bmax > smax[0]
        smax[0] = jnp.where(better, bmax, smax[0])
        sidx[0] = jnp.where(better, bidx, sidx[0])

    @pl.when(i == pl.num_programs(0) - 1)
    def _():
        maxv_ref[0, 0] = smax[0]
        maxi_ref[0, 0] = sidx[0]


def _phase3_body(maxi_ref, maxv_ref, nf_ref, thr_ref, w_any, wout_any,
                 throut_ref, row_ref, sem):
    idx = maxi_ref[0, 0]
    maxv = maxv_ref[0, 0]
    fired = maxv > 0.0
    iota = lax.broadcasted_iota(jnp.int32, (M, 1), 0)
    bump = jnp.where((iota == idx) & fired, 0.005 * maxv, 0.0)
    throut_ref[...] = thr_ref[...] + bump

    @pl.when(fired)
    def _():
        cp_in = pltpu.make_async_copy(w_any.at[pl.ds(idx, 1)], row_ref, sem)
        cp_in.start()
        cp_in.wait()
        w_new = row_ref[...] + (ETA * maxv) * nf_ref[...]
        mean = jnp.sum(w_new) / float(N)
        row_ref[...] = w_new / mean * MU_WTS
        cp_out = pltpu.make_async_copy(row_ref, wout_any.at[pl.ds(idx, 1)],
                                       sem)
        cp_out.start()
        cp_out.wait()


@functools.cache
def _sc_phase1():
    return pl.kernel(
        _sc_phase1_body,
        out_type=jax.ShapeDtypeStruct((N,), jnp.float32),
        mesh=plsc.VectorSubcoreMesh(core_axis_name="c",
                                    subcore_axis_name="s",
                                    num_cores=2, num_subcores=16),
        compiler_params=pltpu.CompilerParams(needs_layout_passes=False),
        scratch_types=[
            pltpu.VMEM((N,), jnp.float32),
            pltpu.VMEM((IDX_CAP,), jnp.int32),
            pltpu.VMEM((SLAB,), jnp.float32),
            pltpu.VMEM((SLAB,), jnp.float32),
            pltpu.VMEM((SLAB,), jnp.float32),
            pltpu.VMEM((SLAB,), jnp.float32),
            pltpu.VMEM((CH, SLAB), jnp.float32),
            pltpu.VMEM((CH, SLAB), jnp.float32),
            pltpu.SemaphoreType.DMA,
            pltpu.SemaphoreType.DMA,
        ],
    )


@jax.jit
def kernel(x, is_firing, retina_weights, lgn_weights, lgn_threshold,
           node_threshold):
    lthr_col = lgn_threshold.reshape(M, 1)
    w2 = retina_weights.reshape(N * NW, SLAB)

    nf = _sc_phase1()(is_firing, x, node_threshold, w2)
    nf_row = nf.reshape(1, N)

    wcopy, lgn_act_col, maxv, maxi = pl.pallas_call(
        _phase2_body,
        grid=(M // LW_BLK,),
        in_specs=[
            pl.BlockSpec((1, N), lambda i: (0, 0)),
            pl.BlockSpec((LW_BLK, N), lambda i: (i, 0)),
            pl.BlockSpec((LW_BLK, 1), lambda i: (i, 0)),
        ],
        out_specs=[
            pl.BlockSpec((LW_BLK, N), lambda i: (i, 0)),
            pl.BlockSpec((LW_BLK, 1), lambda i: (i, 0)),
            pl.BlockSpec(memory_space=pltpu.SMEM),
            pl.BlockSpec(memory_space=pltpu.SMEM),
        ],
        out_shape=[
            jax.ShapeDtypeStruct((M, N), jnp.float32),
            jax.ShapeDtypeStruct((M, 1), jnp.float32),
            jax.ShapeDtypeStruct((1, 1), jnp.float32),
            jax.ShapeDtypeStruct((1, 1), jnp.int32),
        ],
        scratch_shapes=[
            pltpu.SMEM((1,), jnp.float32),
            pltpu.SMEM((1,), jnp.int32),
        ],
    )(nf_row, lgn_weights, lthr_col)

    new_w, new_thr_col = pl.pallas_call(
        _phase3_body,
        grid=(1,),
        in_specs=[
            pl.BlockSpec(memory_space=pltpu.SMEM),
            pl.BlockSpec(memory_space=pltpu.SMEM),
            pl.BlockSpec((1, N), lambda i: (0, 0)),
            pl.BlockSpec((M, 1), lambda i: (0, 0)),
            pl.BlockSpec(memory_space=pl.ANY),
        ],
        out_specs=[
            pl.BlockSpec(memory_space=pl.ANY),
            pl.BlockSpec((M, 1), lambda i: (0, 0)),
        ],
        out_shape=[
            jax.ShapeDtypeStruct((M, N), jnp.float32),
            jax.ShapeDtypeStruct((M, 1), jnp.float32),
        ],
        scratch_shapes=[
            pltpu.VMEM((1, N), jnp.float32),
            pltpu.SemaphoreType.DMA,
        ],
        input_output_aliases={4: 0},
    )(maxi, maxv, nf_row, lthr_col, wcopy)

    return (lgn_act_col.reshape(M), nf.reshape(N), new_w,
            new_thr_col.reshape(M))


# speedup vs baseline: 3.6343x; 3.6343x over previous
import functools

import jax
import jax.numpy as jnp
from jax import lax
from jax.experimental import pallas as pl
from jax.experimental.pallas import tpu as pltpu

N = 4096
M = 1024
ETA = 0.1
MU_WTS = 2.5

RW_BLK = 512
LW_BLK = 128


def _phase1_body(f_ref, x_ref, thr_ref, w_ref, nf_ref):
    nx = lax.dot_general(f_ref[...], w_ref[...],
                         (((1,), (0,)), ((), ())),
                         preferred_element_type=jnp.float32)
    nf_ref[...] = (nx + x_ref[...] > thr_ref[...]).astype(jnp.float32)


def _phase2_body(nf_ref, w_ref, thr_ref, wout_ref, act_ref, maxv_ref,
                 maxi_ref, smax, sidx):
    i = pl.program_id(0)
    w = w_ref[...]
    wout_ref[...] = w
    wb = w.astype(jnp.bfloat16).astype(jnp.float32)
    a = lax.dot_general(wb, nf_ref[...], (((1,), (1,)), ((), ())),
                        preferred_element_type=jnp.float32)
    lgn_act = jnp.maximum(a, 0.0)
    act_ref[...] = lgn_act
    actv = jnp.maximum(lgn_act - thr_ref[...], 0.0)
    bmax = jnp.max(actv)
    iota = lax.broadcasted_iota(jnp.int32, (LW_BLK, 1), 0)
    bidx = jnp.min(jnp.where(actv == bmax, iota, 2 ** 30)) + i * LW_BLK

    @pl.when(i == 0)
    def _():
        smax[0] = bmax
        sidx[0] = bidx

    @pl.when(i > 0)
    def _():
        better = bmax > smax[0]
        smax[0] = jnp.where(better, bmax, smax[0])
        sidx[0] = jnp.where(better, bidx, sidx[0])

    @pl.when(i == pl.num_programs(0) - 1)
    def _():
        maxv_ref[0, 0] = smax[0]
        maxi_ref[0, 0] = sidx[0]


def _phase3_body(maxi_ref, maxv_ref, nf_ref, thr_ref, w_any, wout_any,
                 throut_ref, row_ref, sem):
    idx = maxi_ref[0, 0]
    maxv = maxv_ref[0, 0]
    fired = maxv > 0.0
    iota = lax.broadcasted_iota(jnp.int32, (M, 1), 0)
    bump = jnp.where((iota == idx) & fired, 0.005 * maxv, 0.0)
    throut_ref[...] = thr_ref[...] + bump

    @pl.when(fired)
    def _():
        cp_in = pltpu.make_async_copy(w_any.at[pl.ds(idx, 1)], row_ref, sem)
        cp_in.start()
        cp_in.wait()
        w_new = row_ref[...] + (ETA * maxv) * nf_ref[...]
        mean = jnp.sum(w_new) / float(N)
        row_ref[...] = w_new / mean * MU_WTS
        cp_out = pltpu.make_async_copy(row_ref, wout_any.at[pl.ds(idx, 1)],
                                       sem)
        cp_out.start()
        cp_out.wait()


@jax.jit
def kernel(x, is_firing, retina_weights, lgn_weights, lgn_threshold,
           node_threshold):
    f_row = is_firing.reshape(1, N)
    x_row = x.reshape(1, N)
    nthr_row = node_threshold.reshape(1, N)
    lthr_col = lgn_threshold.reshape(M, 1)

    nf_row = pl.pallas_call(
        _phase1_body,
        grid=(N // RW_BLK,),
        in_specs=[
            pl.BlockSpec((1, N), lambda i: (0, 0)),
            pl.BlockSpec((1, RW_BLK), lambda i: (0, i)),
            pl.BlockSpec((1, RW_BLK), lambda i: (0, i)),
            pl.BlockSpec((N, RW_BLK), lambda i: (0, i)),
        ],
        out_specs=pl.BlockSpec((1, RW_BLK), lambda i: (0, i)),
        out_shape=jax.ShapeDtypeStruct((1, N), jnp.float32),
    )(f_row, x_row, nthr_row, retina_weights)

    wcopy, lgn_act_col, maxv, maxi = pl.pallas_call(
        _phase2_body,
        grid=(M // LW_BLK,),
        in_specs=[
            pl.BlockSpec((1, N), lambda i: (0, 0)),
            pl.BlockSpec((LW_BLK, N), lambda i: (i, 0)),
            pl.BlockSpec((LW_BLK, 1), lambda i: (i, 0)),
        ],
        out_specs=[
            pl.BlockSpec((LW_BLK, N), lambda i: (i, 0)),
            pl.BlockSpec((LW_BLK, 1), lambda i: (i, 0)),
            pl.BlockSpec(memory_space=pltpu.SMEM),
            pl.BlockSpec(memory_space=pltpu.SMEM),
        ],
        out_shape=[
            jax.ShapeDtypeStruct((M, N), jnp.float32),
            jax.ShapeDtypeStruct((M, 1), jnp.float32),
            jax.ShapeDtypeStruct((1, 1), jnp.float32),
            jax.ShapeDtypeStruct((1, 1), jnp.int32),
        ],
        scratch_shapes=[
            pltpu.SMEM((1,), jnp.float32),
            pltpu.SMEM((1,), jnp.int32),
        ],
    )(nf_row, lgn_weights, lthr_col)

    new_w, new_thr_col = pl.pallas_call(
        _phase3_body,
        grid=(1,),
        in_specs=[
            pl.BlockSpec(memory_space=pltpu.SMEM),
            pl.BlockSpec(memory_space=pltpu.SMEM),
            pl.BlockSpec((1, N), lambda i: (0, 0)),
            pl.BlockSpec((M, 1), lambda i: (0, 0)),
            pl.BlockSpec(memory_space=pl.ANY),
        ],
        out_specs=[
            pl.BlockSpec(memory_space=pl.ANY),
            pl.BlockSpec((M, 1), lambda i: (0, 0)),
        ],
        out_shape=[
            jax.ShapeDtypeStruct((M, N), jnp.float32),
            jax.ShapeDtypeStruct((M, 1), jnp.float32),
        ],
        scratch_shapes=[
            pltpu.VMEM((1, N), jnp.float32),
            pltpu.SemaphoreType.DMA,
        ],
        input_output_aliases={4: 0},
    )(maxi, maxv, nf_row, lthr_col, wcopy)

    return (lgn_act_col.reshape(M), nf_row.reshape(N), new_w,
            new_thr_col.reshape(M))
